# Initial kernel scaffold; baseline (speedup 1.0000x reference)
import jax
import jax.numpy as jnp
from jax.experimental import pallas as pl


def kernel(x, neigh_indices, neigh_weights, W0, b0, gamma0, beta0, W1, b1, gamma1, beta1, W2, b2, gamma2, beta2, W_out, b_out):
    raise NotImplementedError("write your pallas kernel here")


# speedup vs baseline: 1.7186x; 1.7186x over previous
import functools

import jax
import jax.numpy as jnp
from jax import lax
from jax.experimental import pallas as pl
from jax.experimental.pallas import tpu as pltpu
from jax.experimental.pallas import tpu_sc as plsc

N_NODES = 10242
BLK = 256
NPAD = ((N_NODES + BLK - 1) // BLK) * BLK
RNB = 75
NUMIDX = RNB * NPAD
GW = 128
NGATH = NUMIDX // GW
EPS = 1e-5
NC, NS = 2, 16
NW = NC * NS
GPW = NGATH // NW
GREM = NGATH - GPW * NW
MAXG = -(-(GPW + GREM) // 8) * 8
NGPAD = (NW - 1) * GPW + MAXG
TDT = jnp.float32


def _sc_gather(table, idx2d):
    mesh = plsc.VectorSubcoreMesh(core_axis_name="c", subcore_axis_name="s")

    @functools.partial(
        pl.kernel,
        mesh=mesh,
        out_type=jax.ShapeDtypeStruct((NUMIDX, 128), table.dtype),
        scratch_types=[
            pltpu.VMEM((MAXG, GW), jnp.int32),
            pltpu.VMEM((GW, 128), table.dtype),
            pltpu.SemaphoreType.DMA,
        ],
    )
    def gk(table_hbm, idx_hbm, out_hbm, idx_s, rows_v, sem):
        wid = lax.axis_index("s") * NC + lax.axis_index("c")
        g0 = wid * GPW
        ng = lax.select(wid == NW - 1, GPW + GREM, GPW)
        pltpu.sync_copy(idx_hbm.at[pl.ds(g0, MAXG)], idx_s)

        @pl.loop(0, ng)
        def _(i):
            pltpu.async_copy(table_hbm.at[idx_s.at[i]], rows_v, sem).wait()
            pltpu.sync_copy(rows_v, out_hbm.at[pl.ds((g0 + i) * GW, GW)])

    return gk(table, idx2d)


def _expand_w(w, ci, cp):
    o = w.shape[0]
    wr = w.reshape(o, 25, ci)
    if cp != ci:
        wr = jnp.pad(wr, ((0, 0), (0, 0), (0, cp - ci)))
    return wr.transpose(1, 2, 0).reshape(25 * cp, o)


def _conv(g, w_tn, we, b, o, cp, interpret=False):
    nblk = NPAD // BLK

    def body(g_ref, w_ref, we_ref, b_ref, y_ref, sm_ref, sq_ref, interp_ref):
        i = pl.program_id(0)
        for k in range(25):
            acc = None
            for j in range(3):
                r = 3 * k + j
                gr = g_ref[r, :, :cp].astype(jnp.float32)
                term = gr * w_ref[:, r : r + 1]
                acc = term if acc is None else acc + term
            interp_ref[:, k * cp : (k + 1) * cp] = acc
        y = jnp.dot(interp_ref[:], we_ref[:],
                    preferred_element_type=jnp.float32) + b_ref[0]
        y_ref[:] = y
        rid = i * BLK + lax.broadcasted_iota(jnp.int32, (BLK, 1), 0)
        ym = jnp.where(rid < N_NODES, y, 0.0)
        ps = jnp.sum(ym, axis=0, keepdims=True)
        ps2 = jnp.sum(ym * ym, axis=0, keepdims=True)

        @pl.when(i == 0)
        def _():
            sm_ref[:] = ps
            sq_ref[:] = ps2

        @pl.when(i > 0)
        def _():
            sm_ref[:] = sm_ref[:] + ps
            sq_ref[:] = sq_ref[:] + ps2

    return pl.pallas_call(
        body,
        grid=(nblk,),
        in_specs=[
            pl.BlockSpec((RNB, BLK, 128), lambda i: (0, i, 0)),
            pl.BlockSpec((BLK, RNB), lambda i: (i, 0)),
            pl.BlockSpec((25 * cp, o), lambda i: (0, 0)),
            pl.BlockSpec((1, o), lambda i: (0, 0)),
        ],
        out_specs=[
            pl.BlockSpec((BLK, o), lambda i: (i, 0)),
            pl.BlockSpec((1, o), lambda i: (0, 0)),
            pl.BlockSpec((1, o), lambda i: (0, 0)),
        ],
        out_shape=[
            jax.ShapeDtypeStruct((NPAD, o), jnp.float32),
            jax.ShapeDtypeStruct((1, o), jnp.float32),
            jax.ShapeDtypeStruct((1, o), jnp.float32),
        ],
        scratch_shapes=[pltpu.VMEM((BLK, 25 * cp), jnp.float32)],
        interpret=interpret,
    )(g, w_tn, we, b)


def _bn_relu(y, sm, sq, gamma, beta, interpret=False):
    o = y.shape[1]
    nb = 2624
    nblk = NPAD // nb
    assert nblk * nb == NPAD

    def body(y_ref, sm_ref, sq_ref, g_ref, be_ref, h_ref):
        mean = sm_ref[0] * (1.0 / N_NODES)
        var = sq_ref[0] * (1.0 / N_NODES) - mean * mean
        s = g_ref[0] * lax.rsqrt(var + EPS)
        t = be_ref[0] - mean * s
        h = jnp.maximum(y_ref[:] * s + t, 0.0).astype(TDT)
        if o == 128:
            h_ref[:] = h
        else:
            h_ref[:, :o] = h
            h_ref[:, o:] = jnp.zeros((nb, 128 - o), TDT)

    return pl.pallas_call(
        body,
        grid=(nblk,),
        in_specs=[
            pl.BlockSpec((nb, o), lambda i: (i, 0)),
            pl.BlockSpec((1, o), lambda i: (0, 0)),
            pl.BlockSpec((1, o), lambda i: (0, 0)),
            pl.BlockSpec((1, o), lambda i: (0, 0)),
            pl.BlockSpec((1, o), lambda i: (0, 0)),
        ],
        out_specs=pl.BlockSpec((nb, 128), lambda i: (i, 0)),
        out_shape=jax.ShapeDtypeStruct((NPAD, 128), TDT),
        interpret=interpret,
    )(y, sm, sq, gamma, beta)


def _forward(x, neigh_indices, neigh_weights, W0, b0, gamma0, beta0, W1, b1,
             gamma1, beta1, W2, b2, gamma2, beta2, W_out, b_out,
             gather_fn, interpret=False):
    r2 = lambda v: v.reshape(1, -1)

    idx_t = jnp.pad(neigh_indices.reshape(N_NODES, RNB).T,
                    ((0, 0), (0, NPAD - N_NODES)))
    idx2d = jnp.pad(idx_t.reshape(NGATH, GW), ((0, NGPAD - NGATH), (0, 0)))
    w_tn = jnp.pad(neigh_weights.reshape(N_NODES, RNB),
                   ((0, NPAD - N_NODES), (0, 0)))

    xp = jnp.pad(x.astype(TDT), ((0, NPAD - N_NODES), (0, 125)))
    g0 = gather_fn(xp, idx2d).reshape(RNB, NPAD, 128)
    y0, sm0, sq0 = _conv(g0, w_tn, _expand_w(W0, 3, 8), r2(b0), 64, 8,
                         interpret)
    h0 = _bn_relu(y0, sm0, sq0, r2(gamma0), r2(beta0), interpret)

    g1 = gather_fn(h0, idx2d).reshape(RNB, NPAD, 128)
    y1, sm1, sq1 = _conv(g1, w_tn, _expand_w(W1, 64, 64), r2(b1), 64, 64,
                         interpret)
    h1 = _bn_relu(y1, sm1, sq1, r2(gamma1), r2(beta1), interpret)

    g2 = gather_fn(h1, idx2d).reshape(RNB, NPAD, 128)
    y2, sm2, sq2 = _conv(g2, w_tn, _expand_w(W2, 64, 64), r2(b2), 128, 64,
                         interpret)
    h2 = _bn_relu(y2, sm2, sq2, r2(gamma2), r2(beta2), interpret)

    g3 = gather_fn(h2, idx2d).reshape(RNB, NPAD, 128)
    y3, _, _ = _conv(g3, w_tn, _expand_w(W_out, 128, 128), r2(b_out), 36, 128,
                     interpret)
    return y3[:N_NODES]


def kernel(x, neigh_indices, neigh_weights, W0, b0, gamma0, beta0, W1, b1,
           gamma1, beta1, W2, b2, gamma2, beta2, W_out, b_out):
    return _forward(x, neigh_indices, neigh_weights, W0, b0, gamma0, beta0,
                    W1, b1, gamma1, beta1, W2, b2, gamma2, beta2, W_out,
                    b_out, _sc_gather)


# speedup vs baseline: 1.7647x; 1.0268x over previous
import functools

import jax
import jax.numpy as jnp
from jax import lax
from jax.experimental import pallas as pl
from jax.experimental.pallas import tpu as pltpu
from jax.experimental.pallas import tpu_sc as plsc

N_NODES = 10242
BLK = 256
NPAD = ((N_NODES + BLK - 1) // BLK) * BLK
RNB = 75
NUMIDX = RNB * NPAD
GW = 128
NGATH = NUMIDX // GW
EPS = 1e-5
NC, NS = 2, 16
NW = NC * NS
GPW = NGATH // NW
GREM = NGATH - GPW * NW
MAXG = -(-(GPW + GREM) // 8) * 8
NGPAD = (NW - 1) * GPW + MAXG
TDT = jnp.float32


def _sc_gather(table, idx2d):
    mesh = plsc.VectorSubcoreMesh(core_axis_name="c", subcore_axis_name="s")
    nbuf = 6

    @functools.partial(
        pl.kernel,
        mesh=mesh,
        out_type=jax.ShapeDtypeStruct((NUMIDX, 128), table.dtype),
        scratch_types=[pltpu.VMEM((MAXG, GW), jnp.int32)]
        + [pltpu.VMEM((GW, 128), table.dtype)] * nbuf
        + [pltpu.SemaphoreType.DMA] * nbuf,
    )
    def gk(table_hbm, idx_hbm, out_hbm, idx_s, *bufs_sems):
        bufs, sems = bufs_sems[:nbuf], bufs_sems[nbuf:]
        wid = lax.axis_index("s") * NC + lax.axis_index("c")
        g0 = wid * GPW
        ng = lax.select(wid == NW - 1, GPW + GREM, GPW)
        pltpu.sync_copy(idx_hbm.at[pl.ds(g0, MAXG)], idx_s)

        @pl.loop(0, ng // nbuf)
        def _(it):
            c = it * nbuf
            cps = [
                pltpu.async_copy(table_hbm.at[idx_s.at[c + b]], bufs[b],
                                 sems[b])
                for b in range(nbuf)
            ]
            for b in range(nbuf):
                cps[b].wait()
                pltpu.sync_copy(bufs[b],
                                out_hbm.at[pl.ds((g0 + c + b) * GW, GW)])

    return gk(table, idx2d)


def _expand_w(w, ci, cp):
    o = w.shape[0]
    wr = w.reshape(o, 25, ci)
    if cp != ci:
        wr = jnp.pad(wr, ((0, 0), (0, 0), (0, cp - ci)))
    return wr.transpose(1, 2, 0).reshape(25 * cp, o)


def _conv(g, w_tn, we, b, o, cp, interpret=False):
    nblk = NPAD // BLK

    def body(g_ref, w_ref, we_ref, b_ref, y_ref, sm_ref, sq_ref, interp_ref):
        i = pl.program_id(0)
        for k in range(25):
            acc = None
            for j in range(3):
                r = 3 * k + j
                gr = g_ref[r, :, :cp].astype(jnp.float32)
                term = gr * w_ref[:, r : r + 1]
                acc = term if acc is None else acc + term
            interp_ref[:, k * cp : (k + 1) * cp] = acc
        y = jnp.dot(interp_ref[:], we_ref[:],
                    preferred_element_type=jnp.float32) + b_ref[0]
        y_ref[:] = y
        rid = i * BLK + lax.broadcasted_iota(jnp.int32, (BLK, 1), 0)
        ym = jnp.where(rid < N_NODES, y, 0.0)
        ps = jnp.sum(ym, axis=0, keepdims=True)
        ps2 = jnp.sum(ym * ym, axis=0, keepdims=True)

        @pl.when(i == 0)
        def _():
            sm_ref[:] = ps
            sq_ref[:] = ps2

        @pl.when(i > 0)
        def _():
            sm_ref[:] = sm_ref[:] + ps
            sq_ref[:] = sq_ref[:] + ps2

    return pl.pallas_call(
        body,
        grid=(nblk,),
        in_specs=[
            pl.BlockSpec((RNB, BLK, 128), lambda i: (0, i, 0)),
            pl.BlockSpec((BLK, RNB), lambda i: (i, 0)),
            pl.BlockSpec((25 * cp, o), lambda i: (0, 0)),
            pl.BlockSpec((1, o), lambda i: (0, 0)),
        ],
        out_specs=[
            pl.BlockSpec((BLK, o), lambda i: (i, 0)),
            pl.BlockSpec((1, o), lambda i: (0, 0)),
            pl.BlockSpec((1, o), lambda i: (0, 0)),
        ],
        out_shape=[
            jax.ShapeDtypeStruct((NPAD, o), jnp.float32),
            jax.ShapeDtypeStruct((1, o), jnp.float32),
            jax.ShapeDtypeStruct((1, o), jnp.float32),
        ],
        scratch_shapes=[pltpu.VMEM((BLK, 25 * cp), jnp.float32)],
        interpret=interpret,
    )(g, w_tn, we, b)


def _bn_relu(y, sm, sq, gamma, beta, interpret=False):
    o = y.shape[1]
    nb = 2624
    nblk = NPAD // nb
    assert nblk * nb == NPAD

    def body(y_ref, sm_ref, sq_ref, g_ref, be_ref, h_ref):
        mean = sm_ref[0] * (1.0 / N_NODES)
        var = sq_ref[0] * (1.0 / N_NODES) - mean * mean
        s = g_ref[0] * lax.rsqrt(var + EPS)
        t = be_ref[0] - mean * s
        h = jnp.maximum(y_ref[:] * s + t, 0.0).astype(TDT)
        if o == 128:
            h_ref[:] = h
        else:
            h_ref[:, :o] = h
            h_ref[:, o:] = jnp.zeros((nb, 128 - o), TDT)

    return pl.pallas_call(
        body,
        grid=(nblk,),
        in_specs=[
            pl.BlockSpec((nb, o), lambda i: (i, 0)),
            pl.BlockSpec((1, o), lambda i: (0, 0)),
            pl.BlockSpec((1, o), lambda i: (0, 0)),
            pl.BlockSpec((1, o), lambda i: (0, 0)),
            pl.BlockSpec((1, o), lambda i: (0, 0)),
        ],
        out_specs=pl.BlockSpec((nb, 128), lambda i: (i, 0)),
        out_shape=jax.ShapeDtypeStruct((NPAD, 128), TDT),
        interpret=interpret,
    )(y, sm, sq, gamma, beta)


def _forward(x, neigh_indices, neigh_weights, W0, b0, gamma0, beta0, W1, b1,
             gamma1, beta1, W2, b2, gamma2, beta2, W_out, b_out,
             gather_fn, interpret=False):
    r2 = lambda v: v.reshape(1, -1)

    idx_t = jnp.pad(neigh_indices.reshape(N_NODES, RNB).T,
                    ((0, 0), (0, NPAD - N_NODES)))
    idx2d = jnp.pad(idx_t.reshape(NGATH, GW), ((0, NGPAD - NGATH), (0, 0)))
    w_tn = jnp.pad(neigh_weights.reshape(N_NODES, RNB),
                   ((0, NPAD - N_NODES), (0, 0)))

    xp = jnp.pad(x.astype(TDT), ((0, NPAD - N_NODES), (0, 125)))
    g0 = gather_fn(xp, idx2d).reshape(RNB, NPAD, 128)
    y0, sm0, sq0 = _conv(g0, w_tn, _expand_w(W0, 3, 8), r2(b0), 64, 8,
                         interpret)
    h0 = _bn_relu(y0, sm0, sq0, r2(gamma0), r2(beta0), interpret)

    g1 = gather_fn(h0, idx2d).reshape(RNB, NPAD, 128)
    y1, sm1, sq1 = _conv(g1, w_tn, _expand_w(W1, 64, 64), r2(b1), 64, 64,
                         interpret)
    h1 = _bn_relu(y1, sm1, sq1, r2(gamma1), r2(beta1), interpret)

    g2 = gather_fn(h1, idx2d).reshape(RNB, NPAD, 128)
    y2, sm2, sq2 = _conv(g2, w_tn, _expand_w(W2, 64, 64), r2(b2), 128, 64,
                         interpret)
    h2 = _bn_relu(y2, sm2, sq2, r2(gamma2), r2(beta2), interpret)

    g3 = gather_fn(h2, idx2d).reshape(RNB, NPAD, 128)
    y3, _, _ = _conv(g3, w_tn, _expand_w(W_out, 128, 128), r2(b_out), 36, 128,
                     interpret)
    return y3[:N_NODES]


def kernel(x, neigh_indices, neigh_weights, W0, b0, gamma0, beta0, W1, b1,
           gamma1, beta1, W2, b2, gamma2, beta2, W_out, b_out):
    return _forward(x, neigh_indices, neigh_weights, W0, b0, gamma0, beta0,
                    W1, b1, gamma1, beta1, W2, b2, gamma2, beta2, W_out,
                    b_out, _sc_gather)


# speedup vs baseline: 1.9737x; 1.1184x over previous
import dataclasses
import functools

import jax
import jax.numpy as jnp
from jax import lax
from jax.experimental import pallas as pl
from jax.experimental.pallas import tpu as pltpu
from jax.experimental.pallas import tpu_sc as plsc

N_NODES = 10242
BLK = 256
NPAD = ((N_NODES + BLK - 1) // BLK) * BLK
RNB = 75
NUMIDX = RNB * NPAD
GW = 128
NGATH = NUMIDX // GW
EPS = 1e-5
NC, NS = 2, 16
NW = NC * NS
GPW = NGATH // NW
GREM = NGATH - GPW * NW
MAXG = -(-(GPW + GREM) // 8) * 8
NGPAD = (NW - 1) * GPW + MAXG
TDT = jnp.float32


def _sc_gather(table, idx2d):
    mesh = plsc.VectorSubcoreMesh(core_axis_name="c", subcore_axis_name="s")
    nbuf = 6

    @functools.partial(
        pl.kernel,
        mesh=mesh,
        out_type=jax.ShapeDtypeStruct((NUMIDX, 128), table.dtype),
        scratch_types=[pltpu.VMEM((MAXG, GW), jnp.int32)]
        + [pltpu.VMEM((GW, 128), table.dtype)] * nbuf
        + [pltpu.SemaphoreType.DMA] * nbuf,
    )
    def gk(table_hbm, idx_hbm, out_hbm, idx_s, *bufs_sems):
        bufs, sems = bufs_sems[:nbuf], bufs_sems[nbuf:]
        wid = lax.axis_index("s") * NC + lax.axis_index("c")
        g0 = wid * GPW
        ng = lax.select(wid == NW - 1, GPW + GREM, GPW)
        pltpu.sync_copy(idx_hbm.at[pl.ds(g0, MAXG)], idx_s)

        @pl.loop(0, ng // nbuf)
        def _(it):
            c = it * nbuf
            cps = [
                pltpu.async_copy(table_hbm.at[idx_s.at[c + b]], bufs[b],
                                 sems[b])
                for b in range(nbuf)
            ]
            for b in range(nbuf):
                cps[b].wait()
                pltpu.sync_copy(bufs[b],
                                out_hbm.at[pl.ds((g0 + c + b) * GW, GW)])

    return gk(table, idx2d)


XFL = 30728
NPW = NPAD // NW
L0C = 8


def _sc_interp0(xflat, idx3, w3):
    mesh = plsc.VectorSubcoreMesh(core_axis_name="c", subcore_axis_name="s")
    nch = NPW // L0C
    cp = pltpu.CompilerParams()
    if "needs_layout_passes" in pltpu.CompilerParams.__dataclass_fields__:
        cp = dataclasses.replace(cp, needs_layout_passes=False)

    @functools.partial(
        pl.kernel,
        mesh=mesh,
        compiler_params=cp,
        out_type=jax.ShapeDtypeStruct((NPAD, 128), jnp.float32),
        scratch_types=[
            pltpu.VMEM((1, XFL), jnp.float32),
            pltpu.VMEM((L0C, 256), jnp.int32),
            pltpu.VMEM((L0C, 256), jnp.float32),
            pltpu.VMEM((L0C, 128), jnp.float32),
            pltpu.SemaphoreType.DMA,
        ],
    )
    def ik(x_hbm, i_hbm, w_hbm, o_hbm, x_v, i_v, w_v, o_v, sem):
        wid = lax.axis_index("s") * NC + lax.axis_index("c")
        n0 = wid * NPW
        pltpu.sync_copy(x_hbm, x_v)

        @pl.loop(0, nch)
        def _(ci):
            base = n0 + ci * L0C
            pltpu.sync_copy(i_hbm.at[pl.ds(base, L0C)], i_v)
            pltpu.sync_copy(w_hbm.at[pl.ds(base, L0C)], w_v)
            for n in range(L0C):
                for v in range(5):
                    acc = None
                    for j in range(3):
                        o = j * 80 + v * 16
                        ivec = i_v[n, pl.ds(o, 16)]
                        term = (w_v[n, pl.ds(o, 16)]
                                * plsc.load_gather(x_v.at[0], [ivec]))
                        acc = term if acc is None else acc + term
                    o_v[n, pl.ds(v * 16, 16)] = acc
                o_v[n, pl.ds(80, 16)] = jnp.zeros((16,), jnp.float32)
                o_v[n, pl.ds(96, 16)] = jnp.zeros((16,), jnp.float32)
                o_v[n, pl.ds(112, 16)] = jnp.zeros((16,), jnp.float32)
            pltpu.sync_copy(o_v, o_hbm.at[pl.ds(base, L0C)])

    return ik(xflat, idx3, w3)


def _l0_streams(neigh_indices, neigh_weights):
    idxr = neigh_indices.reshape(N_NODES, 25, 3)
    nwr = neigh_weights
    cc = jnp.tile(jnp.arange(3, dtype=jnp.int32), 25)
    parts_i, parts_w = [], []
    for j in range(3):
        a = jnp.repeat(idxr[:, :, j], 3, axis=1) * 3 + cc
        w = jnp.repeat(nwr[:, :, j], 3, axis=1)
        parts_i.append(jnp.pad(a, ((0, 0), (0, 5))))
        parts_w.append(jnp.pad(w, ((0, 0), (0, 5))))
    idx3 = jnp.pad(jnp.concatenate(parts_i, axis=1),
                   ((0, NPAD - N_NODES), (0, 16)))
    w3 = jnp.pad(jnp.concatenate(parts_w, axis=1),
                 ((0, NPAD - N_NODES), (0, 16)))
    return idx3, w3


def _matmul0(interp0, we, b, o, interpret=False):
    nblk = NPAD // BLK

    def body(a_ref, we_ref, b_ref, y_ref, sm_ref, sq_ref):
        i = pl.program_id(0)
        y = jnp.dot(a_ref[...], we_ref[...],
                    preferred_element_type=jnp.float32) + b_ref[0]
        y_ref[...] = y
        rid = i * BLK + lax.broadcasted_iota(jnp.int32, (BLK, 1), 0)
        ym = jnp.where(rid < N_NODES, y, 0.0)
        ps = jnp.sum(ym, axis=0, keepdims=True)
        ps2 = jnp.sum(ym * ym, axis=0, keepdims=True)

        @pl.when(i == 0)
        def _():
            sm_ref[...] = ps
            sq_ref[...] = ps2

        @pl.when(i > 0)
        def _():
            sm_ref[...] = sm_ref[...] + ps
            sq_ref[...] = sq_ref[...] + ps2

    return pl.pallas_call(
        body,
        grid=(nblk,),
        in_specs=[
            pl.BlockSpec((BLK, 128), lambda i: (i, 0)),
            pl.BlockSpec((128, o), lambda i: (0, 0)),
            pl.BlockSpec((1, o), lambda i: (0, 0)),
        ],
        out_specs=[
            pl.BlockSpec((BLK, o), lambda i: (i, 0)),
            pl.BlockSpec((1, o), lambda i: (0, 0)),
            pl.BlockSpec((1, o), lambda i: (0, 0)),
        ],
        out_shape=[
            jax.ShapeDtypeStruct((NPAD, o), jnp.float32),
            jax.ShapeDtypeStruct((1, o), jnp.float32),
            jax.ShapeDtypeStruct((1, o), jnp.float32),
        ],
        interpret=interpret,
    )(interp0, we, b)


def _expand_w(w, ci, cp):
    o = w.shape[0]
    wr = w.reshape(o, 25, ci)
    if cp != ci:
        wr = jnp.pad(wr, ((0, 0), (0, 0), (0, cp - ci)))
    return wr.transpose(1, 2, 0).reshape(25 * cp, o)


def _conv(g, w_tn, we, b, o, cp, interpret=False):
    nblk = NPAD // BLK

    def body(g_ref, w_ref, we_ref, b_ref, y_ref, sm_ref, sq_ref, interp_ref):
        i = pl.program_id(0)
        for k in range(25):
            acc = None
            for j in range(3):
                r = 3 * k + j
                gr = g_ref[r, :, :cp].astype(jnp.float32)
                term = gr * w_ref[:, r : r + 1]
                acc = term if acc is None else acc + term
            interp_ref[:, k * cp : (k + 1) * cp] = acc
        y = jnp.dot(interp_ref[:], we_ref[:],
                    preferred_element_type=jnp.float32) + b_ref[0]
        y_ref[:] = y
        rid = i * BLK + lax.broadcasted_iota(jnp.int32, (BLK, 1), 0)
        ym = jnp.where(rid < N_NODES, y, 0.0)
        ps = jnp.sum(ym, axis=0, keepdims=True)
        ps2 = jnp.sum(ym * ym, axis=0, keepdims=True)

        @pl.when(i == 0)
        def _():
            sm_ref[:] = ps
            sq_ref[:] = ps2

        @pl.when(i > 0)
        def _():
            sm_ref[:] = sm_ref[:] + ps
            sq_ref[:] = sq_ref[:] + ps2

    return pl.pallas_call(
        body,
        grid=(nblk,),
        in_specs=[
            pl.BlockSpec((RNB, BLK, 128), lambda i: (0, i, 0)),
            pl.BlockSpec((BLK, RNB), lambda i: (i, 0)),
            pl.BlockSpec((25 * cp, o), lambda i: (0, 0)),
            pl.BlockSpec((1, o), lambda i: (0, 0)),
        ],
        out_specs=[
            pl.BlockSpec((BLK, o), lambda i: (i, 0)),
            pl.BlockSpec((1, o), lambda i: (0, 0)),
            pl.BlockSpec((1, o), lambda i: (0, 0)),
        ],
        out_shape=[
            jax.ShapeDtypeStruct((NPAD, o), jnp.float32),
            jax.ShapeDtypeStruct((1, o), jnp.float32),
            jax.ShapeDtypeStruct((1, o), jnp.float32),
        ],
        scratch_shapes=[pltpu.VMEM((BLK, 25 * cp), jnp.float32)],
        interpret=interpret,
    )(g, w_tn, we, b)


def _bn_relu(y, sm, sq, gamma, beta, interpret=False):
    o = y.shape[1]
    nb = 2624
    nblk = NPAD // nb
    assert nblk * nb == NPAD

    def body(y_ref, sm_ref, sq_ref, g_ref, be_ref, h_ref):
        mean = sm_ref[0] * (1.0 / N_NODES)
        var = sq_ref[0] * (1.0 / N_NODES) - mean * mean
        s = g_ref[0] * lax.rsqrt(var + EPS)
        t = be_ref[0] - mean * s
        h = jnp.maximum(y_ref[:] * s + t, 0.0).astype(TDT)
        if o == 128:
            h_ref[:] = h
        else:
            h_ref[:, :o] = h
            h_ref[:, o:] = jnp.zeros((nb, 128 - o), TDT)

    return pl.pallas_call(
        body,
        grid=(nblk,),
        in_specs=[
            pl.BlockSpec((nb, o), lambda i: (i, 0)),
            pl.BlockSpec((1, o), lambda i: (0, 0)),
            pl.BlockSpec((1, o), lambda i: (0, 0)),
            pl.BlockSpec((1, o), lambda i: (0, 0)),
            pl.BlockSpec((1, o), lambda i: (0, 0)),
        ],
        out_specs=pl.BlockSpec((nb, 128), lambda i: (i, 0)),
        out_shape=jax.ShapeDtypeStruct((NPAD, 128), TDT),
        interpret=interpret,
    )(y, sm, sq, gamma, beta)


def _forward(x, neigh_indices, neigh_weights, W0, b0, gamma0, beta0, W1, b1,
             gamma1, beta1, W2, b2, gamma2, beta2, W_out, b_out,
             gather_fn, interp0_fn, interpret=False):
    r2 = lambda v: v.reshape(1, -1)

    idx_t = jnp.pad(neigh_indices.reshape(N_NODES, RNB).T,
                    ((0, 0), (0, NPAD - N_NODES)))
    idx2d = jnp.pad(idx_t.reshape(NGATH, GW), ((0, NGPAD - NGATH), (0, 0)))
    w_tn = jnp.pad(neigh_weights.reshape(N_NODES, RNB),
                   ((0, NPAD - N_NODES), (0, 0)))

    xflat = jnp.pad(x.reshape(1, -1), ((0, 0), (0, XFL - 3 * N_NODES)))
    idx3, w3 = _l0_streams(neigh_indices, neigh_weights)
    interp0 = interp0_fn(xflat, idx3, w3)
    we0 = jnp.pad(W0.T, ((0, 53), (0, 0)))
    y0, sm0, sq0 = _matmul0(interp0, we0, r2(b0), 64, interpret)
    h0 = _bn_relu(y0, sm0, sq0, r2(gamma0), r2(beta0), interpret)

    g1 = gather_fn(h0, idx2d).reshape(RNB, NPAD, 128)
    y1, sm1, sq1 = _conv(g1, w_tn, _expand_w(W1, 64, 64), r2(b1), 64, 64,
                         interpret)
    h1 = _bn_relu(y1, sm1, sq1, r2(gamma1), r2(beta1), interpret)

    g2 = gather_fn(h1, idx2d).reshape(RNB, NPAD, 128)
    y2, sm2, sq2 = _conv(g2, w_tn, _expand_w(W2, 64, 64), r2(b2), 128, 64,
                         interpret)
    h2 = _bn_relu(y2, sm2, sq2, r2(gamma2), r2(beta2), interpret)

    g3 = gather_fn(h2, idx2d).reshape(RNB, NPAD, 128)
    y3, _, _ = _conv(g3, w_tn, _expand_w(W_out, 128, 128), r2(b_out), 36, 128,
                     interpret)
    return y3[:N_NODES]


def kernel(x, neigh_indices, neigh_weights, W0, b0, gamma0, beta0, W1, b1,
           gamma1, beta1, W2, b2, gamma2, beta2, W_out, b_out):
    return _forward(x, neigh_indices, neigh_weights, W0, b0, gamma0, beta0,
                    W1, b1, gamma1, beta1, W2, b2, gamma2, beta2, W_out,
                    b_out, _sc_gather, _sc_interp0)
